# baseline (speedup 1.0000x reference)
import functools

import jax
import jax.numpy as jnp
from jax import lax
from jax.experimental import pallas as pl
from jax.experimental.pallas import tpu as pltpu
from jax.experimental.pallas import tpu_sc as plsc

B = 16384
D = 64
CTX = 10
NEG = 5
NV = 1 + NEG
ROWS = 1000000
SROWS = ROWS // 2
NC = 2
NS = 16
NW = NC * NS
PERW = B // NW
C = 16
NCHUNK = PERW // C
URPC = C * CTX
VRPC = C * NV

U_GROUPS = [(0, 128), (128, 32)]
V_GROUPS = [(0, 96)]


_mesh = plsc.VectorSubcoreMesh(
    core_axis_name="c", subcore_axis_name="s", num_cores=NC, num_subcores=NS)

@functools.partial(
    pl.kernel,
    out_type=jax.ShapeDtypeStruct((B * NV,), jnp.float32),
    mesh=_mesh,
    scratch_types=[
        pltpu.VMEM((PERW * CTX,), jnp.int32),
        pltpu.VMEM((PERW * NV,), jnp.int32),
        pltpu.VMEM((PERW * CTX,), jnp.int32),
        pltpu.VMEM((PERW * NV,), jnp.int32),
        pltpu.VMEM((PERW * CTX,), jnp.int32),
        pltpu.VMEM((PERW * NV,), jnp.int32),
        pltpu.VMEM((2 * URPC, 2 * D), jnp.float32),
        pltpu.VMEM((2 * VRPC, 2 * D), jnp.float32),
        pltpu.VMEM((PERW * NV,), jnp.float32),
        pltpu.SemaphoreType.DMA,
        pltpu.SemaphoreType.DMA,
    ],
    compiler_params=pltpu.CompilerParams(needs_layout_passes=False),
)
def _sc_scores(uidx_hbm, vidx_hbm, u_pack, v_pack, out_hbm,
               uidx_v, vidx_v, usid_v, vsid_v, uh_v, vh_v,
               ubuf, vbuf, scores, sem0, sem1):
    wid = lax.axis_index("s") * NC + lax.axis_index("c")
    base = wid * PERW
    iota = lax.iota(jnp.int32, 16)
    pltpu.sync_copy(uidx_hbm.at[pl.ds(base * CTX, PERW * CTX)], uidx_v)
    pltpu.sync_copy(vidx_hbm.at[pl.ds(base * NV, PERW * NV)], vidx_v)

    def split(g, _, raw, sid, half, n16):
        del n16
        x = raw[pl.ds(g * 16, 16)]
        hi = x >= SROWS
        sid[pl.ds(g * 16, 16)] = jnp.where(hi, x - SROWS, x)
        half[pl.ds(g * 16, 16)] = jnp.where(hi, D, 0).astype(jnp.int32)
        return _

    lax.fori_loop(0, PERW * CTX // 16,
                  functools.partial(split, raw=uidx_v, sid=usid_v, half=uh_v,
                                    n16=None), 0)
    lax.fori_loop(0, PERW * NV // 16,
                  functools.partial(split, raw=vidx_v, sid=vsid_v, half=vh_v,
                                    n16=None), 0)

    def issue(ch, par, sem):
        for off, n in U_GROUPS:
            pltpu.async_copy(
                u_pack.at[usid_v.at[pl.ds(ch * URPC + off, n)]],
                ubuf.at[pl.ds(par * URPC + off, n)], sem)
        for off, n in V_GROUPS:
            pltpu.async_copy(
                v_pack.at[vsid_v.at[pl.ds(ch * VRPC + off, n)]],
                vbuf.at[pl.ds(par * VRPC + off, n)], sem)

    def drain(ch, par, sem):
        for off, n in U_GROUPS:
            pltpu.make_async_copy(
                u_pack.at[usid_v.at[pl.ds(ch * URPC + off, n)]],
                ubuf.at[pl.ds(par * URPC + off, n)], sem).wait()
        for off, n in V_GROUPS:
            pltpu.make_async_copy(
                v_pack.at[vsid_v.at[pl.ds(ch * VRPC + off, n)]],
                vbuf.at[pl.ds(par * VRPC + off, n)], sem).wait()

    def compute(ch, par):
        urow = [par * URPC + iota * CTX + c for c in range(CTX)]
        ucol = [plsc.load_gather(uh_v, [ch * URPC + iota * CTX + c])
                for c in range(CTX)]
        vrow = [par * VRPC + iota * NV + t for t in range(NV)]
        vcol = [plsc.load_gather(vh_v, [ch * VRPC + iota * NV + t])
                for t in range(NV)]

        def dstep(d, acc):
            us = plsc.load_gather(ubuf, [urow[0], ucol[0] + d])
            for c in range(1, CTX):
                us = us + plsc.load_gather(ubuf, [urow[c], ucol[c] + d])
            return tuple(
                acc[t] + us * plsc.load_gather(vbuf, [vrow[t], vcol[t] + d])
                for t in range(NV))

        zero = jnp.zeros((16,), jnp.float32)
        acc = lax.fori_loop(0, D, dstep, (zero,) * NV)
        for t in range(NV):
            sgn = 1.0 / CTX if t == 0 else -1.0 / CTX
            scores[pl.ds((ch * NV + t) * 16, 16)] = acc[t] * sgn

    issue(0, 0, sem0)

    def pair(i, _):
        ch0 = i * 2
        drain(ch0, 0, sem0)

        @pl.when(i == 0)
        def _first():
            issue(1, 1, sem1)

        @pl.when(ch0 + 2 < NCHUNK)
        def _next0():
            issue(ch0 + 2, 0, sem0)

        compute(ch0, 0)
        drain(ch0 + 1, 1, sem1)

        @pl.when(ch0 + 3 < NCHUNK)
        def _next1():
            issue(ch0 + 3, 1, sem1)

        compute(ch0 + 1, 1)
        return _

    lax.fori_loop(0, NCHUNK // 2, pair, 0)
    pltpu.sync_copy(scores, out_hbm.at[pl.ds(base * NV, PERW * NV)])


def _loss_body(x_ref, o_ref):
    o_ref[0, 0] = -jnp.sum(jax.nn.log_sigmoid(x_ref[...]))


_loss = pl.pallas_call(
    _loss_body,
    out_shape=jax.ShapeDtypeStruct((1, 1), jnp.float32),
    out_specs=pl.BlockSpec(memory_space=pltpu.SMEM),
)


def kernel(batch_0, batch_1, batch_2, u_table, v_table):
    uidx = batch_0.astype(jnp.int32).reshape(B * CTX)
    vidx = jnp.concatenate(
        [batch_1[:, None], batch_2], axis=1).astype(jnp.int32).reshape(B * NV)
    u_pack = jnp.concatenate([u_table[0:SROWS], u_table[SROWS:ROWS]], axis=1)
    v_pack = jnp.concatenate([v_table[0:SROWS], v_table[SROWS:ROWS]], axis=1)
    scores = _sc_scores(uidx, vidx, u_pack, v_pack)
    loss = _loss(scores.reshape(B * NV // 128, 128))
    return loss.reshape(())


# speedup vs baseline: 1.2738x; 1.2738x over previous
import functools

import jax
import jax.numpy as jnp
from jax import lax
from jax.experimental import pallas as pl
from jax.experimental.pallas import tpu as pltpu
from jax.experimental.pallas import tpu_sc as plsc

B = 16384
D = 64
CTX = 10
NEG = 5
NV = 1 + NEG
ROWS = 1000000
SROWS = ROWS // 2
NC = 2
NS = 16
NW = NC * NS
PERW = B // NW
C = 16
NCHUNK = PERW // C
URPC = C * CTX
VRPC = C * NV

U_GROUPS = [(0, 128), (128, 32)]
V_GROUPS = [(0, 96)]


_mesh = plsc.VectorSubcoreMesh(
    core_axis_name="c", subcore_axis_name="s", num_cores=NC, num_subcores=NS)

@functools.partial(
    pl.kernel,
    out_type=jax.ShapeDtypeStruct((B * NV,), jnp.float32),
    mesh=_mesh,
    scratch_types=[
        pltpu.VMEM((PERW * CTX,), jnp.int32),
        pltpu.VMEM((PERW * NV,), jnp.int32),
        pltpu.VMEM((PERW * CTX,), jnp.int32),
        pltpu.VMEM((PERW * NV,), jnp.int32),
        pltpu.VMEM((PERW * CTX,), jnp.int32),
        pltpu.VMEM((PERW * NV,), jnp.int32),
        pltpu.VMEM((2 * URPC, 2 * D), jnp.float32),
        pltpu.VMEM((2 * VRPC, 2 * D), jnp.float32),
        pltpu.VMEM((PERW * NV,), jnp.float32),
        pltpu.SemaphoreType.DMA,
        pltpu.SemaphoreType.DMA,
    ],
    compiler_params=pltpu.CompilerParams(needs_layout_passes=False),
)
def _sc_scores(uidx_hbm, vidx_hbm, u_pack, v_pack, out_hbm,
               uidx_v, vidx_v, usid_v, vsid_v, uh_v, vh_v,
               ubuf, vbuf, scores, sem0, sem1):
    wid = lax.axis_index("s") * NC + lax.axis_index("c")
    base = wid * PERW
    iota = lax.iota(jnp.int32, 16)
    pltpu.sync_copy(uidx_hbm.at[pl.ds(base * CTX, PERW * CTX)], uidx_v)
    pltpu.sync_copy(vidx_hbm.at[pl.ds(base * NV, PERW * NV)], vidx_v)

    def split(g, _, raw, sid, half, n16):
        del n16
        x = raw[pl.ds(g * 16, 16)]
        sid[pl.ds(g * 16, 16)] = x >> 1
        half[pl.ds(g * 16, 16)] = (x & 1) << 6
        return _

    lax.fori_loop(0, PERW * CTX // 16,
                  functools.partial(split, raw=uidx_v, sid=usid_v, half=uh_v,
                                    n16=None), 0)
    lax.fori_loop(0, PERW * NV // 16,
                  functools.partial(split, raw=vidx_v, sid=vsid_v, half=vh_v,
                                    n16=None), 0)

    def issue(ch, par, sem):
        for off, n in U_GROUPS:
            pltpu.async_copy(
                u_pack.at[usid_v.at[pl.ds(ch * URPC + off, n)]],
                ubuf.at[pl.ds(par * URPC + off, n)], sem)
        for off, n in V_GROUPS:
            pltpu.async_copy(
                v_pack.at[vsid_v.at[pl.ds(ch * VRPC + off, n)]],
                vbuf.at[pl.ds(par * VRPC + off, n)], sem)

    def drain(ch, par, sem):
        for off, n in U_GROUPS:
            pltpu.make_async_copy(
                u_pack.at[usid_v.at[pl.ds(ch * URPC + off, n)]],
                ubuf.at[pl.ds(par * URPC + off, n)], sem).wait()
        for off, n in V_GROUPS:
            pltpu.make_async_copy(
                v_pack.at[vsid_v.at[pl.ds(ch * VRPC + off, n)]],
                vbuf.at[pl.ds(par * VRPC + off, n)], sem).wait()

    def compute(ch, par):
        urow = [par * URPC + iota * CTX + c for c in range(CTX)]
        ucol = [plsc.load_gather(uh_v, [ch * URPC + iota * CTX + c])
                for c in range(CTX)]
        vrow = [par * VRPC + iota * NV + t for t in range(NV)]
        vcol = [plsc.load_gather(vh_v, [ch * VRPC + iota * NV + t])
                for t in range(NV)]

        def dstep(d, acc):
            us = plsc.load_gather(ubuf, [urow[0], ucol[0] + d])
            for c in range(1, CTX):
                us = us + plsc.load_gather(ubuf, [urow[c], ucol[c] + d])
            return tuple(
                acc[t] + us * plsc.load_gather(vbuf, [vrow[t], vcol[t] + d])
                for t in range(NV))

        zero = jnp.zeros((16,), jnp.float32)
        acc = lax.fori_loop(0, D, dstep, (zero,) * NV)
        for t in range(NV):
            sgn = 1.0 / CTX if t == 0 else -1.0 / CTX
            scores[pl.ds((ch * NV + t) * 16, 16)] = acc[t] * sgn

    issue(0, 0, sem0)

    def pair(i, _):
        ch0 = i * 2
        drain(ch0, 0, sem0)

        @pl.when(i == 0)
        def _first():
            issue(1, 1, sem1)

        @pl.when(ch0 + 2 < NCHUNK)
        def _next0():
            issue(ch0 + 2, 0, sem0)

        compute(ch0, 0)
        drain(ch0 + 1, 1, sem1)

        @pl.when(ch0 + 3 < NCHUNK)
        def _next1():
            issue(ch0 + 3, 1, sem1)

        compute(ch0 + 1, 1)
        return _

    lax.fori_loop(0, NCHUNK // 2, pair, 0)
    pltpu.sync_copy(scores, out_hbm.at[pl.ds(base * NV, PERW * NV)])


def _loss_body(x_ref, o_ref):
    o_ref[0, 0] = -jnp.sum(jax.nn.log_sigmoid(x_ref[...]))


_loss = pl.pallas_call(
    _loss_body,
    out_shape=jax.ShapeDtypeStruct((1, 1), jnp.float32),
    out_specs=pl.BlockSpec(memory_space=pltpu.SMEM),
)


def kernel(batch_0, batch_1, batch_2, u_table, v_table):
    uidx = batch_0.astype(jnp.int32).reshape(B * CTX)
    vidx = jnp.concatenate(
        [batch_1[:, None], batch_2], axis=1).astype(jnp.int32).reshape(B * NV)
    u_pack = u_table[0:ROWS].reshape(SROWS, 2 * D)
    v_pack = v_table[0:ROWS].reshape(SROWS, 2 * D)
    scores = _sc_scores(uidx, vidx, u_pack, v_pack)
    loss = _loss(scores.reshape(B * NV // 128, 128))
    return loss.reshape(())
